# baseline (speedup 1.0000x reference)
import functools

import jax
import jax.numpy as jnp
from jax import lax
from jax.experimental import pallas as pl
from jax.experimental.pallas import tpu as pltpu
from jax.experimental.pallas import tpu_sc as plsc

D_MODEL = 768
D_FF = 768
E = 64
K = 2
T = 2048
TILE = 128
NT = 96
GROUP = 4
NT2 = NT // GROUP
R_MAX = NT * TILE
NT_PAD = 128
NC, NS = 2, 16
NW = NC * NS
CHUNK = (T * K) // NW
TOK = T // NW
LANES = 16


def _routing_body(x_ref, gw_ref, gb_ref, r_ref, w0_ref, w1_ref, eot_ref,
                  tix2_ref, mf_ref, pos_ref):
    xv = x_ref[...]
    logits = jnp.dot(xv, gw_ref[...], preferred_element_type=jnp.float32)
    logits = logits + gb_ref[...]
    eidx = lax.broadcasted_iota(jnp.int32, (T, E), 1)

    m1 = jnp.max(logits, axis=1, keepdims=True)
    a1 = jnp.min(jnp.where(logits == m1, eidx, E), axis=1, keepdims=True)
    masked = jnp.where(eidx == a1, -jnp.inf, logits)
    m2 = jnp.max(masked, axis=1, keepdims=True)
    a2 = jnp.min(jnp.where(masked == m2, eidx, E), axis=1, keepdims=True)

    w0 = 1.0 / (1.0 + jnp.exp(m2 - m1))
    w1 = 1.0 - w0

    mf = ((eidx == a1) | (eidx == a2)).astype(jnp.float32)
    mf_ref[...] = mf

    R = 256
    ti = lax.broadcasted_iota(jnp.int32, (R, R), 0)
    tj = lax.broadcasted_iota(jnp.int32, (R, R), 1)
    tril = (tj < ti).astype(jnp.float32)

    def body(i, carry):
        blk = mf_ref[pl.ds(i * R, R), :]
        pos_ref[pl.ds(i * R, R), :] = (
            jnp.dot(tril, blk, preferred_element_type=jnp.float32) + carry)
        return carry + jnp.sum(blk, axis=0, keepdims=True)

    counts = lax.fori_loop(0, T // R, body, jnp.zeros((1, E), jnp.float32))

    pc = jnp.ceil(counts / float(TILE)) * float(TILE)
    ei = lax.broadcasted_iota(jnp.int32, (E, E), 0)
    ej = lax.broadcasted_iota(jnp.int32, (E, E), 1)
    triu = (ei < ej).astype(jnp.float32)
    off = jnp.dot(pc, triu, preferred_element_type=jnp.float32)

    tgt = off + pos_ref[...]
    r0 = jnp.sum(jnp.where(eidx == a1, tgt, 0.0), axis=1, keepdims=True)
    r1 = jnp.sum(jnp.where(eidx == a2, tgt, 0.0), axis=1, keepdims=True)
    r_ref[...] = jnp.concatenate(
        [r0.astype(jnp.int32), r1.astype(jnp.int32)], axis=1)

    w0_ref[...] = jnp.broadcast_to(w0, (T, LANES))
    w1_ref[...] = jnp.broadcast_to(w1, (T, LANES))

    ends = off + pc
    eye = (ei == ej).astype(jnp.float32)
    ends_col = jnp.sum(jnp.broadcast_to(ends, (E, E)) * eye, axis=1,
                       keepdims=True)
    nact = jnp.sum(pc, axis=1, keepdims=True) / float(TILE)
    tstart = (lax.broadcasted_iota(jnp.int32, (E, NT_PAD), 1)
              .astype(jnp.float32) * float(TILE))
    tstart = jnp.minimum(tstart, (nact - 1.0) * float(TILE))
    cmp = (ends_col <= tstart).astype(jnp.int32)
    eot = jnp.sum(cmp, axis=0, keepdims=True)
    eot_ref[...] = jnp.minimum(eot, E - 1)
    nact2 = jnp.ceil(nact / float(GROUP)).astype(jnp.int32)
    ti = lax.broadcasted_iota(jnp.int32, (1, NT_PAD), 1)
    tix2_ref[...] = jnp.minimum(ti, nact2 - 1)


_routing_call = pl.pallas_call(
    _routing_body,
    out_shape=[
        jax.ShapeDtypeStruct((T, K), jnp.int32),
        jax.ShapeDtypeStruct((T, LANES), jnp.float32),
        jax.ShapeDtypeStruct((T, LANES), jnp.float32),
        jax.ShapeDtypeStruct((1, NT_PAD), jnp.int32),
        jax.ShapeDtypeStruct((1, NT_PAD), jnp.int32),
    ],
    scratch_shapes=[
        pltpu.VMEM((T, E), jnp.float32),
        pltpu.VMEM((T, E), jnp.float32),
    ],
)


def _ffn_body(eot_ref, tix2_ref, x_ref, *refs):
    del eot_ref
    y_ref = refs[-1]
    wrefs = refs[:-1]

    @pl.when(pl.program_id(0) <= tix2_ref[NT_PAD - 1])
    def _():
        for g in range(GROUP):
            w1_ref, b1_ref, w2_ref, b2_ref = wrefs[4 * g:4 * g + 4]
            xg = x_ref[g * TILE:(g + 1) * TILE]
            h = jnp.dot(xg, w1_ref[0], preferred_element_type=jnp.float32)
            h = jnp.maximum(h + b1_ref[0], 0.0)
            y = jnp.dot(h, w2_ref[0], preferred_element_type=jnp.float32)
            y_ref[g * TILE:(g + 1) * TILE] = y + b2_ref[0]


def _wspecs(g):
    return [
        pl.BlockSpec((1, D_MODEL, D_FF),
                     lambda i, eot, tix2: (eot[GROUP * i + g], 0, 0)),
        pl.BlockSpec((1, 1, D_FF),
                     lambda i, eot, tix2: (eot[GROUP * i + g], 0, 0)),
        pl.BlockSpec((1, D_FF, D_MODEL),
                     lambda i, eot, tix2: (eot[GROUP * i + g], 0, 0)),
        pl.BlockSpec((1, 1, D_MODEL),
                     lambda i, eot, tix2: (eot[GROUP * i + g], 0, 0)),
    ]


_ffn_call = pl.pallas_call(
    _ffn_body,
    grid_spec=pltpu.PrefetchScalarGridSpec(
        num_scalar_prefetch=2,
        grid=(NT2,),
        in_specs=(
            [pl.BlockSpec((GROUP * TILE, D_MODEL),
                          lambda i, eot, tix2: (tix2[i], 0))]
            + [s for g in range(GROUP) for s in _wspecs(g)]
        ),
        out_specs=pl.BlockSpec((GROUP * TILE, D_MODEL),
                               lambda i, eot, tix2: (tix2[i], 0)),
    ),
    out_shape=jax.ShapeDtypeStruct((R_MAX, D_MODEL), jnp.float32),
)


@functools.lru_cache(maxsize=1)
def _sc_kernels():
    mesh = plsc.VectorSubcoreMesh(
        core_axis_name="c", subcore_axis_name="s",
        num_cores=NC, num_subcores=NS)

    @functools.partial(
        pl.kernel,
        out_type=jax.ShapeDtypeStruct((R_MAX, D_MODEL), jnp.float32),
        mesh=mesh,
        scratch_types=[
            pltpu.VMEM((CHUNK,), jnp.int32),
            pltpu.VMEM((CHUNK, D_MODEL), jnp.float32),
            pltpu.SemaphoreType.DMA,
        ],
    )
    def dispatch(x_hbm, ridx_hbm, xs_hbm, idx_v, xbuf, sem):
        wid = lax.axis_index("s") * NC + lax.axis_index("c")
        base = pl.multiple_of(wid * CHUNK, CHUNK)
        pltpu.sync_copy(ridx_hbm.at[pl.ds(base, CHUNK)], idx_v)
        tok = pl.multiple_of(jnp.bitwise_and(base, T - 1), CHUNK)
        pltpu.sync_copy(x_hbm.at[pl.ds(tok, CHUNK)], xbuf)
        pltpu.async_copy(xbuf, xs_hbm.at[idx_v], sem).wait()

    @functools.partial(
        pl.kernel,
        out_type=jax.ShapeDtypeStruct((T, D_MODEL), jnp.float32),
        mesh=mesh,
        scratch_types=[
            pltpu.VMEM((TOK,), jnp.int32),
            pltpu.VMEM((TOK,), jnp.int32),
            pltpu.VMEM((TOK, D_MODEL), jnp.float32),
            pltpu.VMEM((TOK, D_MODEL), jnp.float32),
            pltpu.VMEM((TOK, LANES), jnp.float32),
            pltpu.VMEM((TOK, LANES), jnp.float32),
            pltpu.SemaphoreType.DMA,
            pltpu.SemaphoreType.DMA,
        ],
    )
    def combine(y_hbm, ridx_hbm, w0_hbm, w1_hbm, out_hbm,
                i0, i1, b0, b1, wb0, wb1, s0, s1):
        wid = lax.axis_index("s") * NC + lax.axis_index("c")
        base = pl.multiple_of(wid * TOK, TOK)
        pltpu.sync_copy(ridx_hbm.at[pl.ds(base, TOK)], i0)
        pltpu.sync_copy(ridx_hbm.at[pl.ds(T + base, TOK)], i1)
        pltpu.sync_copy(w0_hbm.at[pl.ds(base, TOK)], wb0)
        pltpu.sync_copy(w1_hbm.at[pl.ds(base, TOK)], wb1)
        c0 = pltpu.async_copy(y_hbm.at[i0], b0, s0)
        c1 = pltpu.async_copy(y_hbm.at[i1], b1, s1)
        c0.wait()
        c1.wait()

        def row(j, carry):
            wv0 = wb0[j, pl.ds(0, LANES)]
            wv1 = wb1[j, pl.ds(0, LANES)]
            for c in range(D_MODEL // LANES):
                sl = pl.ds(c * LANES, LANES)
                b0[j, sl] = wv0 * b0[j, sl] + wv1 * b1[j, sl]
            return carry

        lax.fori_loop(0, TOK, row, 0)
        pltpu.sync_copy(b0, out_hbm.at[pl.ds(base, TOK)])

    return dispatch, combine


@jax.jit
def _moe(x, gate_W, gate_b, W1, b1, W2, b2):
    x2d = x.reshape(T, D_MODEL)
    dispatch, combine = _sc_kernels()
    r, w0b, w1b, eot, tix2 = _routing_call(x2d, gate_W, gate_b.reshape(1, E))
    ridx = jnp.concatenate([r[:, 0], r[:, 1]], axis=0)
    x_sorted = dispatch(x2d, ridx)
    b1r = b1.reshape(E, 1, D_FF)
    b2r = b2.reshape(E, 1, D_MODEL)
    wargs = [W1, b1r, W2, b2r] * GROUP
    y_sorted = _ffn_call(eot.reshape(NT_PAD), tix2.reshape(NT_PAD), x_sorted,
                         *wargs)
    out = combine(y_sorted, ridx, w0b, w1b)
    return out.reshape(x.shape)


def kernel(x, gate_W, gate_b, W1, b1, W2, b2):
    return _moe(x, gate_W, gate_b, W1, b1, W2, b2)


# speedup vs baseline: 1.0128x; 1.0128x over previous
import functools

import jax
import jax.numpy as jnp
from jax import lax
from jax.experimental import pallas as pl
from jax.experimental.pallas import tpu as pltpu
from jax.experimental.pallas import tpu_sc as plsc

D_MODEL = 768
D_FF = 768
E = 64
K = 2
T = 2048
TILE = 128
NT = 96
GROUP = 2
NT2 = NT // GROUP
R_MAX = NT * TILE
NT_PAD = 128
NC, NS = 2, 16
NW = NC * NS
CHUNK = (T * K) // NW
TOK = T // NW
LANES = 16


def _routing_body(x_ref, gw_ref, gb_ref, r_ref, w0_ref, w1_ref, eot_ref,
                  tix2_ref, mf_ref, pos_ref):
    xv = x_ref[...]
    logits = jnp.dot(xv, gw_ref[...], preferred_element_type=jnp.float32)
    logits = logits + gb_ref[...]
    eidx = lax.broadcasted_iota(jnp.int32, (T, E), 1)

    m1 = jnp.max(logits, axis=1, keepdims=True)
    a1 = jnp.min(jnp.where(logits == m1, eidx, E), axis=1, keepdims=True)
    masked = jnp.where(eidx == a1, -jnp.inf, logits)
    m2 = jnp.max(masked, axis=1, keepdims=True)
    a2 = jnp.min(jnp.where(masked == m2, eidx, E), axis=1, keepdims=True)

    w0 = 1.0 / (1.0 + jnp.exp(m2 - m1))
    w1 = 1.0 - w0

    mf = ((eidx == a1) | (eidx == a2)).astype(jnp.float32)
    mf_ref[...] = mf

    R = 256
    ti = lax.broadcasted_iota(jnp.int32, (R, R), 0)
    tj = lax.broadcasted_iota(jnp.int32, (R, R), 1)
    tril = (tj < ti).astype(jnp.float32)

    def body(i, carry):
        blk = mf_ref[pl.ds(i * R, R), :]
        pos_ref[pl.ds(i * R, R), :] = (
            jnp.dot(tril, blk, preferred_element_type=jnp.float32) + carry)
        return carry + jnp.sum(blk, axis=0, keepdims=True)

    counts = lax.fori_loop(0, T // R, body, jnp.zeros((1, E), jnp.float32))

    pc = jnp.ceil(counts / float(TILE)) * float(TILE)
    ei = lax.broadcasted_iota(jnp.int32, (E, E), 0)
    ej = lax.broadcasted_iota(jnp.int32, (E, E), 1)
    triu = (ei < ej).astype(jnp.float32)
    off = jnp.dot(pc, triu, preferred_element_type=jnp.float32)

    tgt = off + pos_ref[...]
    r0 = jnp.sum(jnp.where(eidx == a1, tgt, 0.0), axis=1, keepdims=True)
    r1 = jnp.sum(jnp.where(eidx == a2, tgt, 0.0), axis=1, keepdims=True)
    r_ref[...] = jnp.concatenate(
        [r0.astype(jnp.int32), r1.astype(jnp.int32)], axis=1)

    w0_ref[...] = jnp.broadcast_to(w0, (T, LANES))
    w1_ref[...] = jnp.broadcast_to(w1, (T, LANES))

    ends = off + pc
    eye = (ei == ej).astype(jnp.float32)
    ends_col = jnp.sum(jnp.broadcast_to(ends, (E, E)) * eye, axis=1,
                       keepdims=True)
    nact = jnp.sum(pc, axis=1, keepdims=True) / float(TILE)
    tstart = (lax.broadcasted_iota(jnp.int32, (E, NT_PAD), 1)
              .astype(jnp.float32) * float(TILE))
    tstart = jnp.minimum(tstart, (nact - 1.0) * float(TILE))
    cmp = (ends_col <= tstart).astype(jnp.int32)
    eot = jnp.sum(cmp, axis=0, keepdims=True)
    eot_ref[...] = jnp.minimum(eot, E - 1)
    nact2 = jnp.ceil(nact / float(GROUP)).astype(jnp.int32)
    ti = lax.broadcasted_iota(jnp.int32, (1, NT_PAD), 1)
    tix2_ref[...] = jnp.minimum(ti, nact2 - 1)


_routing_call = pl.pallas_call(
    _routing_body,
    out_shape=[
        jax.ShapeDtypeStruct((T, K), jnp.int32),
        jax.ShapeDtypeStruct((T, LANES), jnp.float32),
        jax.ShapeDtypeStruct((T, LANES), jnp.float32),
        jax.ShapeDtypeStruct((1, NT_PAD), jnp.int32),
        jax.ShapeDtypeStruct((1, NT_PAD), jnp.int32),
    ],
    scratch_shapes=[
        pltpu.VMEM((T, E), jnp.float32),
        pltpu.VMEM((T, E), jnp.float32),
    ],
)


def _ffn_body(eot_ref, tix2_ref, x_ref, *refs):
    del eot_ref
    y_ref = refs[-1]
    wrefs = refs[:-1]

    @pl.when(pl.program_id(0) <= tix2_ref[NT_PAD - 1])
    def _():
        for g in range(GROUP):
            w1_ref, b1_ref, w2_ref, b2_ref = wrefs[4 * g:4 * g + 4]
            xg = x_ref[g * TILE:(g + 1) * TILE]
            h = jnp.dot(xg, w1_ref[0], preferred_element_type=jnp.float32)
            h = jnp.maximum(h + b1_ref[0], 0.0)
            y = jnp.dot(h, w2_ref[0], preferred_element_type=jnp.float32)
            y_ref[g * TILE:(g + 1) * TILE] = y + b2_ref[0]


def _wspecs(g):
    return [
        pl.BlockSpec((1, D_MODEL, D_FF),
                     lambda i, eot, tix2: (eot[GROUP * i + g], 0, 0)),
        pl.BlockSpec((1, 1, D_FF),
                     lambda i, eot, tix2: (eot[GROUP * i + g], 0, 0)),
        pl.BlockSpec((1, D_FF, D_MODEL),
                     lambda i, eot, tix2: (eot[GROUP * i + g], 0, 0)),
        pl.BlockSpec((1, 1, D_MODEL),
                     lambda i, eot, tix2: (eot[GROUP * i + g], 0, 0)),
    ]


_ffn_call = pl.pallas_call(
    _ffn_body,
    grid_spec=pltpu.PrefetchScalarGridSpec(
        num_scalar_prefetch=2,
        grid=(NT2,),
        in_specs=(
            [pl.BlockSpec((GROUP * TILE, D_MODEL),
                          lambda i, eot, tix2: (tix2[i], 0))]
            + [s for g in range(GROUP) for s in _wspecs(g)]
        ),
        out_specs=pl.BlockSpec((GROUP * TILE, D_MODEL),
                               lambda i, eot, tix2: (tix2[i], 0)),
    ),
    out_shape=jax.ShapeDtypeStruct((R_MAX, D_MODEL), jnp.float32),
)


@functools.lru_cache(maxsize=1)
def _sc_kernels():
    mesh = plsc.VectorSubcoreMesh(
        core_axis_name="c", subcore_axis_name="s",
        num_cores=NC, num_subcores=NS)

    @functools.partial(
        pl.kernel,
        out_type=jax.ShapeDtypeStruct((R_MAX, D_MODEL), jnp.float32),
        mesh=mesh,
        scratch_types=[
            pltpu.VMEM((CHUNK,), jnp.int32),
            pltpu.VMEM((CHUNK, D_MODEL), jnp.float32),
            pltpu.SemaphoreType.DMA,
        ],
    )
    def dispatch(x_hbm, ridx_hbm, xs_hbm, idx_v, xbuf, sem):
        wid = lax.axis_index("s") * NC + lax.axis_index("c")
        base = pl.multiple_of(wid * CHUNK, CHUNK)
        pltpu.sync_copy(ridx_hbm.at[pl.ds(base, CHUNK)], idx_v)
        tok = pl.multiple_of(jnp.bitwise_and(base, T - 1), CHUNK)
        pltpu.sync_copy(x_hbm.at[pl.ds(tok, CHUNK)], xbuf)
        pltpu.async_copy(xbuf, xs_hbm.at[idx_v], sem).wait()

    @functools.partial(
        pl.kernel,
        out_type=jax.ShapeDtypeStruct((T, D_MODEL), jnp.float32),
        mesh=mesh,
        scratch_types=[
            pltpu.VMEM((TOK,), jnp.int32),
            pltpu.VMEM((TOK,), jnp.int32),
            pltpu.VMEM((TOK, D_MODEL), jnp.float32),
            pltpu.VMEM((TOK, D_MODEL), jnp.float32),
            pltpu.VMEM((TOK, LANES), jnp.float32),
            pltpu.VMEM((TOK, LANES), jnp.float32),
            pltpu.SemaphoreType.DMA,
            pltpu.SemaphoreType.DMA,
        ],
    )
    def combine(y_hbm, ridx_hbm, w0_hbm, w1_hbm, out_hbm,
                i0, i1, b0, b1, wb0, wb1, s0, s1):
        wid = lax.axis_index("s") * NC + lax.axis_index("c")
        base = pl.multiple_of(wid * TOK, TOK)
        pltpu.sync_copy(ridx_hbm.at[pl.ds(base, TOK)], i0)
        pltpu.sync_copy(ridx_hbm.at[pl.ds(T + base, TOK)], i1)
        pltpu.sync_copy(w0_hbm.at[pl.ds(base, TOK)], wb0)
        pltpu.sync_copy(w1_hbm.at[pl.ds(base, TOK)], wb1)
        c0 = pltpu.async_copy(y_hbm.at[i0], b0, s0)
        c1 = pltpu.async_copy(y_hbm.at[i1], b1, s1)
        c0.wait()
        c1.wait()

        def row(j, carry):
            wv0 = wb0[j, pl.ds(0, LANES)]
            wv1 = wb1[j, pl.ds(0, LANES)]
            for c in range(D_MODEL // LANES):
                sl = pl.ds(c * LANES, LANES)
                b0[j, sl] = wv0 * b0[j, sl] + wv1 * b1[j, sl]
            return carry

        lax.fori_loop(0, TOK, row, 0)
        pltpu.sync_copy(b0, out_hbm.at[pl.ds(base, TOK)])

    return dispatch, combine


@jax.jit
def _moe(x, gate_W, gate_b, W1, b1, W2, b2):
    x2d = x.reshape(T, D_MODEL)
    dispatch, combine = _sc_kernels()
    r, w0b, w1b, eot, tix2 = _routing_call(x2d, gate_W, gate_b.reshape(1, E))
    ridx = jnp.concatenate([r[:, 0], r[:, 1]], axis=0)
    x_sorted = dispatch(x2d, ridx)
    b1r = b1.reshape(E, 1, D_FF)
    b2r = b2.reshape(E, 1, D_MODEL)
    wargs = [W1, b1r, W2, b2r] * GROUP
    y_sorted = _ffn_call(eot.reshape(NT_PAD), tix2.reshape(NT_PAD), x_sorted,
                         *wargs)
    out = combine(y_sorted, ridx, w0b, w1b)
    return out.reshape(x.shape)


def kernel(x, gate_W, gate_b, W1, b1, W2, b2):
    return _moe(x, gate_W, gate_b, W1, b1, W2, b2)


# speedup vs baseline: 1.0185x; 1.0057x over previous
import functools

import jax
import jax.numpy as jnp
from jax import lax
from jax.experimental import pallas as pl
from jax.experimental.pallas import tpu as pltpu
from jax.experimental.pallas import tpu_sc as plsc

D_MODEL = 768
D_FF = 768
E = 64
K = 2
T = 2048
TILE = 128
NT = 96
GROUP = 2
NT2 = NT // GROUP
R_MAX = NT * TILE
NT_PAD = 128
NC, NS = 2, 16
NW = NC * NS
CHUNK = (T * K) // NW
TOK = T // NW
LANES = 16


def _routing_body(x_ref, gw_ref, gb_ref, r_ref, w0_ref, w1_ref, eot_ref,
                  tix2_ref, mf_ref, pos_ref):
    xv = x_ref[...]
    logits = jnp.dot(xv, gw_ref[...], preferred_element_type=jnp.float32)
    logits = logits + gb_ref[...]
    eidx = lax.broadcasted_iota(jnp.int32, (T, E), 1)

    m1 = jnp.max(logits, axis=1, keepdims=True)
    a1 = jnp.min(jnp.where(logits == m1, eidx, E), axis=1, keepdims=True)
    masked = jnp.where(eidx == a1, -jnp.inf, logits)
    m2 = jnp.max(masked, axis=1, keepdims=True)
    a2 = jnp.min(jnp.where(masked == m2, eidx, E), axis=1, keepdims=True)

    w0 = 1.0 / (1.0 + jnp.exp(m2 - m1))
    w1 = 1.0 - w0

    mf = ((eidx == a1) | (eidx == a2)).astype(jnp.float32)
    mf_ref[...] = mf

    R = 256
    ti = lax.broadcasted_iota(jnp.int32, (R, R), 0)
    tj = lax.broadcasted_iota(jnp.int32, (R, R), 1)
    tril = (tj < ti).astype(jnp.float32)

    def body(i, carry):
        blk = mf_ref[pl.ds(i * R, R), :]
        pos_ref[pl.ds(i * R, R), :] = (
            jnp.dot(tril, blk, preferred_element_type=jnp.float32) + carry)
        return carry + jnp.sum(blk, axis=0, keepdims=True)

    counts = lax.fori_loop(0, T // R, body, jnp.zeros((1, E), jnp.float32))

    pc = jnp.ceil(counts / float(TILE)) * float(TILE)
    ei = lax.broadcasted_iota(jnp.int32, (E, E), 0)
    ej = lax.broadcasted_iota(jnp.int32, (E, E), 1)
    triu = (ei < ej).astype(jnp.float32)
    off = jnp.dot(pc, triu, preferred_element_type=jnp.float32)

    tgt = off + pos_ref[...]
    r0 = jnp.sum(jnp.where(eidx == a1, tgt, 0.0), axis=1, keepdims=True)
    r1 = jnp.sum(jnp.where(eidx == a2, tgt, 0.0), axis=1, keepdims=True)
    r_ref[...] = jnp.concatenate(
        [r0.astype(jnp.int32), r1.astype(jnp.int32)], axis=1)

    w0_ref[...] = jnp.broadcast_to(w0, (T, LANES))
    w1_ref[...] = jnp.broadcast_to(w1, (T, LANES))

    ends = off + pc
    eye = (ei == ej).astype(jnp.float32)
    ends_col = jnp.sum(jnp.broadcast_to(ends, (E, E)) * eye, axis=1,
                       keepdims=True)
    nact = jnp.sum(pc, axis=1, keepdims=True) / float(TILE)
    tstart = (lax.broadcasted_iota(jnp.int32, (E, NT_PAD), 1)
              .astype(jnp.float32) * float(TILE))
    tstart = jnp.minimum(tstart, (nact - 1.0) * float(TILE))
    cmp = (ends_col <= tstart).astype(jnp.int32)
    eot = jnp.sum(cmp, axis=0, keepdims=True)
    eot_ref[...] = jnp.minimum(eot, E - 1)
    nact2 = jnp.ceil(nact / float(GROUP)).astype(jnp.int32)
    ti = lax.broadcasted_iota(jnp.int32, (1, NT_PAD), 1)
    tix2_ref[...] = jnp.minimum(ti, nact2 - 1)


_routing_call = pl.pallas_call(
    _routing_body,
    out_shape=[
        jax.ShapeDtypeStruct((T, K), jnp.int32),
        jax.ShapeDtypeStruct((T, LANES), jnp.float32),
        jax.ShapeDtypeStruct((T, LANES), jnp.float32),
        jax.ShapeDtypeStruct((1, NT_PAD), jnp.int32),
        jax.ShapeDtypeStruct((1, NT_PAD), jnp.int32),
    ],
    scratch_shapes=[
        pltpu.VMEM((T, E), jnp.float32),
        pltpu.VMEM((T, E), jnp.float32),
    ],
)


def _ffn_body(eot_ref, tix2_ref, x_ref, *refs):
    del eot_ref
    y_ref = refs[-1]
    wrefs = refs[:-1]

    @pl.when(pl.program_id(0) <= tix2_ref[NT_PAD - 1])
    def _():
        for g in range(GROUP):
            w1_ref, b1_ref, w2_ref, b2_ref = wrefs[4 * g:4 * g + 4]
            xg = x_ref[g * TILE:(g + 1) * TILE]
            h = jnp.dot(xg, w1_ref[0], preferred_element_type=jnp.float32)
            h = jnp.maximum(h + b1_ref[0], 0.0)
            y = jnp.dot(h, w2_ref[0], preferred_element_type=jnp.float32)
            y_ref[g * TILE:(g + 1) * TILE] = y + b2_ref[0]


def _wspecs(g):
    return [
        pl.BlockSpec((1, D_MODEL, D_FF),
                     lambda i, eot, tix2: (eot[GROUP * i + g], 0, 0)),
        pl.BlockSpec((1, 1, D_FF),
                     lambda i, eot, tix2: (eot[GROUP * i + g], 0, 0)),
        pl.BlockSpec((1, D_FF, D_MODEL),
                     lambda i, eot, tix2: (eot[GROUP * i + g], 0, 0)),
        pl.BlockSpec((1, 1, D_MODEL),
                     lambda i, eot, tix2: (eot[GROUP * i + g], 0, 0)),
    ]


_ffn_call = pl.pallas_call(
    _ffn_body,
    grid_spec=pltpu.PrefetchScalarGridSpec(
        num_scalar_prefetch=2,
        grid=(NT2,),
        in_specs=(
            [pl.BlockSpec((GROUP * TILE, D_MODEL),
                          lambda i, eot, tix2: (tix2[i], 0))]
            + [s for g in range(GROUP) for s in _wspecs(g)]
        ),
        out_specs=pl.BlockSpec((GROUP * TILE, D_MODEL),
                               lambda i, eot, tix2: (tix2[i], 0)),
    ),
    out_shape=jax.ShapeDtypeStruct((R_MAX, D_MODEL), jnp.float32),
)


@functools.lru_cache(maxsize=1)
def _sc_kernels():
    mesh = plsc.VectorSubcoreMesh(
        core_axis_name="c", subcore_axis_name="s",
        num_cores=NC, num_subcores=NS)

    @functools.partial(
        pl.kernel,
        out_type=jax.ShapeDtypeStruct((R_MAX, D_MODEL), jnp.float32),
        mesh=mesh,
        scratch_types=[
            pltpu.VMEM((CHUNK,), jnp.int32),
            pltpu.VMEM((CHUNK, D_MODEL), jnp.float32),
            pltpu.SemaphoreType.DMA,
        ],
    )
    def dispatch(x_hbm, ridx_hbm, xs_hbm, idx_v, xbuf, sem):
        wid = lax.axis_index("s") * NC + lax.axis_index("c")
        base = pl.multiple_of(wid * CHUNK, CHUNK)
        pltpu.sync_copy(ridx_hbm.at[pl.ds(base, CHUNK)], idx_v)
        tok = pl.multiple_of(jnp.bitwise_and(base, T - 1), CHUNK)
        pltpu.sync_copy(x_hbm.at[pl.ds(tok, CHUNK)], xbuf)
        pltpu.async_copy(xbuf, xs_hbm.at[idx_v], sem).wait()

    CCH = 16
    NCH = TOK // CCH

    @functools.partial(
        pl.kernel,
        out_type=jax.ShapeDtypeStruct((T, D_MODEL), jnp.float32),
        mesh=mesh,
        scratch_types=[
            pltpu.VMEM((TOK,), jnp.int32),
            pltpu.VMEM((TOK,), jnp.int32),
            pltpu.VMEM((TOK, D_MODEL), jnp.float32),
            pltpu.VMEM((TOK, D_MODEL), jnp.float32),
            pltpu.VMEM((TOK, LANES), jnp.float32),
            pltpu.VMEM((TOK, LANES), jnp.float32),
            pltpu.SemaphoreType.DMA,
            pltpu.SemaphoreType.DMA,
            pltpu.SemaphoreType.DMA,
        ],
    )
    def combine(y_hbm, ridx_hbm, w0_hbm, w1_hbm, out_hbm,
                i0, i1, b0, b1, wb0, wb1, s0, s1, so):
        wid = lax.axis_index("s") * NC + lax.axis_index("c")
        base = pl.multiple_of(wid * TOK, TOK)
        pltpu.sync_copy(ridx_hbm.at[pl.ds(base, TOK)], i0)
        pltpu.sync_copy(ridx_hbm.at[pl.ds(T + base, TOK)], i1)
        pltpu.sync_copy(w0_hbm.at[pl.ds(base, TOK)], wb0)
        pltpu.sync_copy(w1_hbm.at[pl.ds(base, TOK)], wb1)
        g0, g1 = [], []
        for c in range(NCH):
            sl = pl.ds(c * CCH, CCH)
            g0.append(pltpu.async_copy(y_hbm.at[i0.at[sl]], b0.at[sl], s0))
            g1.append(pltpu.async_copy(y_hbm.at[i1.at[sl]], b1.at[sl], s1))
        outs = []
        for c in range(NCH):
            g0[c].wait()
            g1[c].wait()

            def row(j, carry):
                wv0 = wb0[j, pl.ds(0, LANES)]
                wv1 = wb1[j, pl.ds(0, LANES)]
                for q in range(D_MODEL // LANES):
                    sq = pl.ds(q * LANES, LANES)
                    b0[j, sq] = wv0 * b0[j, sq] + wv1 * b1[j, sq]
                return carry

            lax.fori_loop(c * CCH, (c + 1) * CCH, row, 0)
            sl = pl.ds(c * CCH, CCH)
            outs.append(pltpu.async_copy(
                b0.at[sl], out_hbm.at[pl.ds(base + c * CCH, CCH)], so))
        for o in outs:
            o.wait()

    return dispatch, combine


@jax.jit
def _moe(x, gate_W, gate_b, W1, b1, W2, b2):
    x2d = x.reshape(T, D_MODEL)
    dispatch, combine = _sc_kernels()
    r, w0b, w1b, eot, tix2 = _routing_call(x2d, gate_W, gate_b.reshape(1, E))
    ridx = jnp.concatenate([r[:, 0], r[:, 1]], axis=0)
    x_sorted = dispatch(x2d, ridx)
    b1r = b1.reshape(E, 1, D_FF)
    b2r = b2.reshape(E, 1, D_MODEL)
    wargs = [W1, b1r, W2, b2r] * GROUP
    y_sorted = _ffn_call(eot.reshape(NT_PAD), tix2.reshape(NT_PAD), x_sorted,
                         *wargs)
    out = combine(y_sorted, ridx, w0b, w1b)
    return out.reshape(x.shape)


def kernel(x, gate_W, gate_b, W1, b1, W2, b2):
    return _moe(x, gate_W, gate_b, W1, b1, W2, b2)
